# baseline (speedup 1.0000x reference)
import functools

import jax
import jax.numpy as jnp
from jax import lax
from jax.experimental import pallas as pl
from jax.experimental.pallas import tpu as pltpu
from jax.experimental.pallas import tpu_sc as plsc

_DIM = 33
_NLUT = _DIM * _DIM * _DIM
_NLUT_PAD = 35944
_BR = 8
_BC = 128
_LANES = 16


def _sc_lut_apply(lut_pad, x):
    nbatch, _, nrows, ncols = x.shape
    info = plsc.get_sparse_core_info()
    nw = info.num_cores * info.num_subcores
    cblk = ncols // _BC
    blocks_per_plane = (nrows // _BR) * cblk
    bpt = blocks_per_plane // nw
    nchunks = nbatch * bpt

    inv_binsize = jnp.float32((_DIM - 1) / 1.000001)
    mesh = plsc.VectorSubcoreMesh(core_axis_name="c", subcore_axis_name="s")

    @functools.partial(
        pl.kernel,
        mesh=mesh,
        compiler_params=pltpu.CompilerParams(needs_layout_passes=False),
        out_type=jax.ShapeDtypeStruct(x.shape, jnp.float32),
        scratch_types=[
            pltpu.VMEM((_NLUT_PAD,), jnp.int32),
            pltpu.VMEM((_NLUT_PAD,), jnp.int32),
            pltpu.VMEM((_NLUT_PAD,), jnp.int32),
        ] + [pltpu.VMEM((_BR, _BC), jnp.float32)] * 12 + [
            pltpu.SemaphoreType.DMA,
            pltpu.SemaphoreType.DMA,
            pltpu.SemaphoreType.DMA,
            pltpu.SemaphoreType.DMA,
        ],
    )
    def sc_kernel(lut_hbm, x_hbm, out_hbm, lut0, lut1, lut2,
                  r0, g0, b0, r1, g1, b1, p0, q0, u0, p1, q1, u1,
                  sem_i0, sem_i1, sem_o0, sem_o1):
        wid = lax.axis_index("s") * info.num_cores + lax.axis_index("c")
        pltpu.sync_copy(lut_hbm.at[pl.ds(0, _NLUT_PAD)], lut0)
        pltpu.sync_copy(lut_hbm.at[pl.ds(_NLUT_PAD, _NLUT_PAD)], lut1)
        pltpu.sync_copy(lut_hbm.at[pl.ds(2 * _NLUT_PAD, _NLUT_PAD)], lut2)
        in_sems = (sem_i0, sem_i1)
        out_sems = (sem_o0, sem_o1)
        in_bufs = ((r0, g0, b0), (r1, g1, b1))
        out_bufs = ((p0, q0, u0), (p1, q1, u1))

        def block_pos(ci):
            bi = lax.shift_right_logical(ci, 3)
            j = jnp.bitwise_and(ci, bpt - 1)
            pos = wid * bpt + j
            row0 = pl.multiple_of(
                lax.shift_left(lax.shift_right_logical(pos, 2), 3), _BR)
            col0 = pl.multiple_of(
                lax.shift_left(jnp.bitwise_and(pos, cblk - 1), 7), _BC)
            return bi, row0, col0

        def issue_in(ci, slot):
            bi, row0, col0 = block_pos(jnp.minimum(ci, nchunks - 1))
            for c in range(3):
                pltpu.async_copy(
                    x_hbm.at[bi, c, pl.ds(row0, _BR), pl.ds(col0, _BC)],
                    in_bufs[slot][c], in_sems[slot])

        def wait_in(slot):
            for c in range(3):
                pltpu.make_async_copy(
                    x_hbm.at[0, 0, pl.ds(0, _BR), pl.ds(0, _BC)],
                    in_bufs[slot][c], in_sems[slot]).wait()

        def issue_out(ci, slot):
            bi, row0, col0 = block_pos(ci)
            for c in range(3):
                pltpu.async_copy(
                    out_bufs[slot][c],
                    out_hbm.at[bi, c, pl.ds(row0, _BR), pl.ds(col0, _BC)],
                    out_sems[slot])

        def wait_out(slot):
            for c in range(3):
                pltpu.make_async_copy(
                    out_bufs[slot][c],
                    out_hbm.at[0, 0, pl.ds(0, _BR), pl.ds(0, _BC)],
                    out_sems[slot]).wait()

        def compute(slot):
            rv, gv, bv = in_bufs[slot]
            o0, o1, o2 = out_bufs[slot]

            @plsc.parallel_loop(0, _BR * _BC // _LANES, unroll=8)
            def vbody(i):
                row = lax.shift_right_logical(i, 3)
                s = pl.ds(lax.shift_left(jnp.bitwise_and(i, 7), 4), _LANES)
                rq = rv[row, s] * inv_binsize
                gq = gv[row, s] * inv_binsize
                bq = bv[row, s] * inv_binsize
                rid = rq.astype(jnp.int32)
                gid = gq.astype(jnp.int32)
                bid = bq.astype(jnp.int32)
                rd = rq - rid.astype(jnp.float32)
                gd = gq - gid.astype(jnp.float32)
                bd = bq - bid.astype(jnp.float32)
                base = bid * (_DIM * _DIM) + gid * _DIM + rid
                wrp = plsc.pack(1.0 - rd, rd, format=plsc.PackFormat.INTERLEAVED)
                u = gd * bd
                w10 = gd - u
                w01 = bd - u
                w00 = (1.0 - gd) - w01
                a0 = jnp.zeros((2 * _LANES,), jnp.bfloat16)
                a1 = jnp.zeros((2 * _LANES,), jnp.bfloat16)
                a2 = jnp.zeros((2 * _LANES,), jnp.bfloat16)
                for dg, db, wgb in ((0, 0, w00), (0, 1, w01),
                                    (1, 0, w10), (1, 1, u)):
                    wp = plsc.pack(wgb, wgb, format=plsc.PackFormat.INTERLEAVED)
                    off = db * (_DIM * _DIM) + dg * _DIM
                    idx = base + off if off else base
                    a0 = a0 + wp * plsc.bitcast(
                        plsc.load_gather(lut0, [idx]), jnp.bfloat16)
                    a1 = a1 + wp * plsc.bitcast(
                        plsc.load_gather(lut1, [idx]), jnp.bfloat16)
                    a2 = a2 + wp * plsc.bitcast(
                        plsc.load_gather(lut2, [idx]), jnp.bfloat16)
                e0, d0 = plsc.unpack(a0 * wrp, format=plsc.PackFormat.INTERLEAVED)
                e1, d1 = plsc.unpack(a1 * wrp, format=plsc.PackFormat.INTERLEAVED)
                e2, d2 = plsc.unpack(a2 * wrp, format=plsc.PackFormat.INTERLEAVED)
                o0[row, s] = e0 + d0
                o1[row, s] = e1 + d1
                o2[row, s] = e2 + d2

        issue_in(0, 0)

        def pair_body(k, _):
            for half in range(2):
                ci = 2 * k + half
                issue_in(ci + 1, 1 - half)
                wait_in(half)
                pl.when(k >= 1)(lambda: wait_out(half))
                compute(half)
                issue_out(ci, half)
            return 0

        lax.fori_loop(0, nchunks // 2, pair_body, 0)
        wait_out(0)
        wait_out(1)
        wait_in(0)

    return sc_kernel(lut_pad, x)


def kernel(LUT, x):
    lut3 = LUT.reshape(3, _NLUT)
    nxt = jnp.concatenate(
        [lut3[:, 1:], jnp.zeros((3, 1), jnp.float32)], axis=1)
    lo = lax.bitcast_convert_type(
        lut3.astype(jnp.bfloat16), jnp.uint16).astype(jnp.uint32)
    hi = lax.bitcast_convert_type(
        nxt.astype(jnp.bfloat16), jnp.uint16).astype(jnp.uint32)
    packed = lax.bitcast_convert_type(
        lo | (hi << jnp.uint32(16)), jnp.int32)
    lut_pad = jnp.pad(
        packed, ((0, 0), (0, _NLUT_PAD - _NLUT))).reshape(-1)
    return _sc_lut_apply(lut_pad, x)


# speedup vs baseline: 1.1084x; 1.1084x over previous
import functools

import jax
import jax.numpy as jnp
from jax import lax
from jax.experimental import pallas as pl
from jax.experimental.pallas import tpu as pltpu
from jax.experimental.pallas import tpu_sc as plsc

_DIM = 33
_RS = 32
_NLUT = _DIM * _DIM * _RS
_BR = 8
_BC = 256
_LANES = 16


def _sc_lut_apply(lut_packed, x):
    nbatch, _, nrows, ncols = x.shape
    info = plsc.get_sparse_core_info()
    nw = info.num_cores * info.num_subcores
    cblk = ncols // _BC
    blocks_per_plane = (nrows // _BR) * cblk
    bpt = blocks_per_plane // nw
    nchunks = nbatch * bpt

    inv_binsize = jnp.float32((_DIM - 1) / 1.000001)
    mesh = plsc.VectorSubcoreMesh(core_axis_name="c", subcore_axis_name="s")

    @functools.partial(
        pl.kernel,
        mesh=mesh,
        compiler_params=pltpu.CompilerParams(needs_layout_passes=False),
        out_type=jax.ShapeDtypeStruct(x.shape, jnp.float32),
        scratch_types=[
            pltpu.VMEM((_NLUT,), jnp.int32),
            pltpu.VMEM((_NLUT,), jnp.int32),
            pltpu.VMEM((_NLUT,), jnp.int32),
        ] + [pltpu.VMEM((_BR, _BC), jnp.float32)] * 12 + [
            pltpu.SemaphoreType.DMA,
            pltpu.SemaphoreType.DMA,
            pltpu.SemaphoreType.DMA,
            pltpu.SemaphoreType.DMA,
        ],
    )
    def sc_kernel(lut_hbm, x_hbm, out_hbm, lut0, lut1, lut2,
                  r0, g0, b0, r1, g1, b1, p0, q0, u0, p1, q1, u1,
                  sem_i0, sem_i1, sem_o0, sem_o1):
        wid = lax.axis_index("s") * info.num_cores + lax.axis_index("c")
        pltpu.sync_copy(lut_hbm.at[pl.ds(0, _NLUT)], lut0)
        pltpu.sync_copy(lut_hbm.at[pl.ds(_NLUT, _NLUT)], lut1)
        pltpu.sync_copy(lut_hbm.at[pl.ds(2 * _NLUT, _NLUT)], lut2)
        in_sems = (sem_i0, sem_i1)
        out_sems = (sem_o0, sem_o1)
        in_bufs = ((r0, g0, b0), (r1, g1, b1))
        out_bufs = ((p0, q0, u0), (p1, q1, u1))

        def block_pos(ci):
            bi = lax.shift_right_logical(ci, 2)
            j = jnp.bitwise_and(ci, bpt - 1)
            pos = wid * bpt + j
            row0 = pl.multiple_of(
                lax.shift_left(lax.shift_right_logical(pos, 1), 3), _BR)
            col0 = pl.multiple_of(
                lax.shift_left(jnp.bitwise_and(pos, cblk - 1), 8), _BC)
            return bi, row0, col0

        def issue_in(ci, slot):
            bi, row0, col0 = block_pos(jnp.minimum(ci, nchunks - 1))
            for c in range(3):
                pltpu.async_copy(
                    x_hbm.at[bi, c, pl.ds(row0, _BR), pl.ds(col0, _BC)],
                    in_bufs[slot][c], in_sems[slot])

        def wait_in(slot):
            for c in range(3):
                pltpu.make_async_copy(
                    x_hbm.at[0, 0, pl.ds(0, _BR), pl.ds(0, _BC)],
                    in_bufs[slot][c], in_sems[slot]).wait()

        def issue_out(ci, slot):
            bi, row0, col0 = block_pos(ci)
            for c in range(3):
                pltpu.async_copy(
                    out_bufs[slot][c],
                    out_hbm.at[bi, c, pl.ds(row0, _BR), pl.ds(col0, _BC)],
                    out_sems[slot])

        def wait_out(slot):
            for c in range(3):
                pltpu.make_async_copy(
                    out_bufs[slot][c],
                    out_hbm.at[0, 0, pl.ds(0, _BR), pl.ds(0, _BC)],
                    out_sems[slot]).wait()

        def compute(slot):
            rv, gv, bv = in_bufs[slot]
            o0, o1, o2 = out_bufs[slot]

            @plsc.parallel_loop(0, _BR * _BC // _LANES, unroll=4)
            def vbody(i):
                row = lax.shift_right_logical(i, 4)
                s = pl.ds(lax.shift_left(jnp.bitwise_and(i, 15), 4), _LANES)
                rq = rv[row, s] * inv_binsize
                gq = gv[row, s] * inv_binsize
                bq = bv[row, s] * inv_binsize
                rid = rq.astype(jnp.int32)
                gid = gq.astype(jnp.int32)
                bid = bq.astype(jnp.int32)
                rd = rq - rid.astype(jnp.float32)
                gd = gq - gid.astype(jnp.float32)
                bd = bq - bid.astype(jnp.float32)
                base = bid * (_DIM * _RS) + gid * _RS + rid
                wrp = plsc.pack(1.0 - rd, rd, format=plsc.PackFormat.INTERLEAVED)
                u = gd * bd
                w10 = gd - u
                w01 = bd - u
                w00 = (1.0 - gd) - w01
                a0 = jnp.zeros((2 * _LANES,), jnp.bfloat16)
                a1 = jnp.zeros((2 * _LANES,), jnp.bfloat16)
                a2 = jnp.zeros((2 * _LANES,), jnp.bfloat16)
                for dg, db, wgb in ((0, 0, w00), (0, 1, w01),
                                    (1, 0, w10), (1, 1, u)):
                    wp = plsc.pack(wgb, wgb, format=plsc.PackFormat.INTERLEAVED)
                    off = db * (_DIM * _RS) + dg * _RS
                    idx = base + off if off else base
                    a0 = a0 + wp * plsc.bitcast(
                        plsc.load_gather(lut0, [idx]), jnp.bfloat16)
                    a1 = a1 + wp * plsc.bitcast(
                        plsc.load_gather(lut1, [idx]), jnp.bfloat16)
                    a2 = a2 + wp * plsc.bitcast(
                        plsc.load_gather(lut2, [idx]), jnp.bfloat16)
                e0, d0 = plsc.unpack(a0 * wrp, format=plsc.PackFormat.INTERLEAVED)
                e1, d1 = plsc.unpack(a1 * wrp, format=plsc.PackFormat.INTERLEAVED)
                e2, d2 = plsc.unpack(a2 * wrp, format=plsc.PackFormat.INTERLEAVED)
                o0[row, s] = e0 + d0
                o1[row, s] = e1 + d1
                o2[row, s] = e2 + d2

        issue_in(0, 0)

        def pair_body(k, _):
            for half in range(2):
                ci = 2 * k + half
                issue_in(ci + 1, 1 - half)
                wait_in(half)
                pl.when(k >= 1)(lambda: wait_out(half))
                compute(half)
                issue_out(ci, half)
            return 0

        lax.fori_loop(0, nchunks // 2, pair_body, 0)
        wait_out(0)
        wait_out(1)
        wait_in(0)

    return sc_kernel(lut_packed, x)


def kernel(LUT, x):
    lut4 = LUT.reshape(3, _DIM, _DIM, _DIM)
    lo4 = lut4[:, :, :, : _RS]
    hi4 = lut4[:, :, :, 1:]
    lo = lax.bitcast_convert_type(
        lo4.astype(jnp.bfloat16), jnp.uint16).astype(jnp.uint32)
    hi = lax.bitcast_convert_type(
        hi4.astype(jnp.bfloat16), jnp.uint16).astype(jnp.uint32)
    packed = lax.bitcast_convert_type(
        lo | (hi << jnp.uint32(16)), jnp.int32)
    return _sc_lut_apply(packed.reshape(-1), x)
